# Initial kernel scaffold; baseline (speedup 1.0000x reference)
import jax
import jax.numpy as jnp
from jax.experimental import pallas as pl


def kernel(x, fm):
    raise NotImplementedError("write your pallas kernel here")


# speedup vs baseline: 1.1077x; 1.1077x over previous
import functools
import math

import jax
import jax.numpy as jnp
from jax import lax
from jax.experimental import pallas as pl
from jax.experimental.pallas import tpu as pltpu, tpu_sc as plsc

_FDIM = 64
_GRID = 513
_LANES = 16
_NC = 2
_NS = 16
_NW = _NC * _NS
_CHUNK = 128


def _sc_body(chunks_per_w, xg_hbm, yg_hbm, tab_hbm, out_hbm,
             xbuf, ybuf, idxbuf, wbuf, rows, outbuf, sem):
    cid = lax.axis_index("c")
    sid = lax.axis_index("s")
    wid = sid * _NC + cid

    def chunk_body(g, _):
        base = (wid * chunks_per_w + g) * _CHUNK
        pltpu.sync_copy(xg_hbm.at[pl.ds(base, _CHUNK)], xbuf)
        pltpu.sync_copy(yg_hbm.at[pl.ds(base, _CHUNK)], ybuf)

        for j in range(_CHUNK // _LANES):
            sl = pl.ds(_LANES * j, _LANES)
            gx = xbuf[sl]
            gy = ybuf[sl]
            ix = ((gx + 1.0) * float(_GRID) - 1.0) * 0.5
            iy = ((gy + 1.0) * float(_GRID) - 1.0) * 0.5
            x0 = (ix + 1.0).astype(jnp.int32) - 1
            y0 = (iy + 1.0).astype(jnp.int32) - 1
            wx1 = ix - x0.astype(jnp.float32)
            wx0 = 1.0 - wx1
            wy1 = iy - y0.astype(jnp.float32)
            wy0 = 1.0 - wy1
            wx0 = jnp.where(x0 >= 0, wx0, 0.0)
            wx1 = jnp.where(x0 <= _GRID - 2, wx1, 0.0)
            wy0 = jnp.where(y0 >= 0, wy0, 0.0)
            wy1 = jnp.where(y0 <= _GRID - 2, wy1, 0.0)
            xc0 = jnp.maximum(x0, 0)
            xc1 = jnp.minimum(x0 + 1, _GRID - 1)
            r0 = jnp.maximum(y0, 0) * _GRID
            r1 = jnp.minimum(y0 + 1, _GRID - 1) * _GRID
            idxbuf[0, sl] = r0 + xc0
            idxbuf[1, sl] = r0 + xc1
            idxbuf[2, sl] = r1 + xc0
            idxbuf[3, sl] = r1 + xc1
            wbuf[0, sl] = wx0 * wy0
            wbuf[1, sl] = wx1 * wy0
            wbuf[2, sl] = wx0 * wy1
            wbuf[3, sl] = wx1 * wy1

        cps = [pltpu.async_copy(tab_hbm.at[idxbuf.at[cc]], rows.at[cc], sem)
               for cc in range(4)]
        for cp in cps:
            cp.wait()

        def blend(g2, _):
            i0 = _LANES * g2
            wv = [wbuf[cc, pl.ds(i0, _LANES)] for cc in range(4)]
            for l in range(_LANES):
                i = i0 + l
                w0, w1, w2, w3 = wv[0][l], wv[1][l], wv[2][l], wv[3][l]
                for k in range(_FDIM // _LANES):
                    ks = pl.ds(_LANES * k, _LANES)
                    outbuf[i, ks] = (rows[0, i, ks] * w0 + rows[1, i, ks] * w1
                                     + rows[2, i, ks] * w2 + rows[3, i, ks] * w3)
            return 0

        lax.fori_loop(0, _CHUNK // _LANES, blend, 0)

        pltpu.sync_copy(outbuf, out_hbm.at[pl.ds(base, _CHUNK)])
        return 0

    lax.fori_loop(0, chunks_per_w, chunk_body, 0)


@functools.partial(jax.jit, static_argnames=("chunks_per_w",))
def _sc_sample(xg, yg, table, chunks_per_w):
    n_pad = chunks_per_w * _NW * _CHUNK
    mesh = plsc.VectorSubcoreMesh(core_axis_name="c", subcore_axis_name="s")
    return pl.kernel(
        functools.partial(_sc_body, chunks_per_w),
        out_type=jax.ShapeDtypeStruct((n_pad, _FDIM), jnp.float32),
        mesh=mesh,
        compiler_params=pltpu.CompilerParams(use_tc_tiling_on_sc=False),
        scratch_types=[
            pltpu.VMEM((_CHUNK,), jnp.float32),
            pltpu.VMEM((_CHUNK,), jnp.float32),
            pltpu.VMEM((4, _CHUNK), jnp.int32),
            pltpu.VMEM((4, _CHUNK), jnp.float32),
            pltpu.VMEM((4, _CHUNK, _FDIM), jnp.float32),
            pltpu.VMEM((_CHUNK, _FDIM), jnp.float32),
            pltpu.SemaphoreType.DMA,
        ],
    )(xg, yg, table)


def kernel(x, fm):
    n = x.shape[0]
    chunks_per_w = math.ceil(n / (_NW * _CHUNK))
    n_pad = chunks_per_w * _NW * _CHUNK
    table = fm[0].reshape(_FDIM, _GRID * _GRID).T
    xp = jnp.pad(x, ((0, n_pad - n), (0, 0)))
    out = _sc_sample(xp[:, 0], xp[:, 1], table, chunks_per_w)
    return out[:n]


# speedup vs baseline: 1.2709x; 1.1474x over previous
import functools
import math

import jax
import jax.numpy as jnp
from jax import lax
from jax.experimental import pallas as pl
from jax.experimental.pallas import tpu as pltpu, tpu_sc as plsc

_FDIM = 64
_GRID = 513
_LANES = 16
_NC = 2
_NS = 16
_NW = _NC * _NS
_CHUNK = 128


def _sc_body(chunks_per_w, xg_hbm, yg_hbm, tab_hbm, out_hbm,
             xbuf, ybuf, idxbuf, wbuf, rows, outbuf,
             gsem0, gsem1, osem0, osem1):
    cid = lax.axis_index("c")
    sid = lax.axis_index("s")
    wid = sid * _NC + cid
    first = wid * chunks_per_w
    gsem = (gsem0, gsem1)
    osem = (osem0, osem1)

    def stage(p, g):
        base = (first + g) * _CHUNK
        pltpu.sync_copy(xg_hbm.at[pl.ds(base, _CHUNK)], xbuf.at[p])
        pltpu.sync_copy(yg_hbm.at[pl.ds(base, _CHUNK)], ybuf.at[p])
        for j in range(_CHUNK // _LANES):
            sl = pl.ds(_LANES * j, _LANES)
            gx = xbuf[p, sl]
            gy = ybuf[p, sl]
            ix = ((gx + 1.0) * float(_GRID) - 1.0) * 0.5
            iy = ((gy + 1.0) * float(_GRID) - 1.0) * 0.5
            x0 = (ix + 1.0).astype(jnp.int32) - 1
            y0 = (iy + 1.0).astype(jnp.int32) - 1
            wx1 = ix - x0.astype(jnp.float32)
            wx0 = 1.0 - wx1
            wy1 = iy - y0.astype(jnp.float32)
            wy0 = 1.0 - wy1
            wx0 = jnp.where(x0 >= 0, wx0, 0.0)
            wx1 = jnp.where(x0 <= _GRID - 2, wx1, 0.0)
            wy0 = jnp.where(y0 >= 0, wy0, 0.0)
            wy1 = jnp.where(y0 <= _GRID - 2, wy1, 0.0)
            xc0 = jnp.maximum(x0, 0)
            xc1 = jnp.minimum(x0 + 1, _GRID - 1)
            r0 = jnp.maximum(y0, 0) * _GRID
            r1 = jnp.minimum(y0 + 1, _GRID - 1) * _GRID
            idxbuf[p, 0, sl] = r0 + xc0
            idxbuf[p, 1, sl] = r0 + xc1
            idxbuf[p, 2, sl] = r1 + xc0
            idxbuf[p, 3, sl] = r1 + xc1
            wbuf[p, 0, sl] = wx0 * wy0
            wbuf[p, 1, sl] = wx1 * wy0
            wbuf[p, 2, sl] = wx0 * wy1
            wbuf[p, 3, sl] = wx1 * wy1
        for cc in range(4):
            pltpu.async_copy(tab_hbm.at[idxbuf.at[p, cc]], rows.at[p, cc],
                             gsem[p])

    def finish(p, g, drain):
        base = (first + g) * _CHUNK
        for cc in range(4):
            pltpu.make_async_copy(tab_hbm.at[idxbuf.at[p, cc]],
                                  rows.at[p, cc], gsem[p]).wait()

        if drain:
            pltpu.make_async_copy(outbuf.at[p],
                                  out_hbm.at[pl.ds(base, _CHUNK)],
                                  osem[p]).wait()

        def blend(g2, _):
            i0 = _LANES * g2
            wv = [wbuf[p, cc, pl.ds(i0, _LANES)] for cc in range(4)]
            for l in range(_LANES):
                i = i0 + l
                w0, w1, w2, w3 = wv[0][l], wv[1][l], wv[2][l], wv[3][l]
                for k in range(_FDIM // _LANES):
                    ks = pl.ds(_LANES * k, _LANES)
                    outbuf[p, i, ks] = (
                        rows[p, 0, i, ks] * w0 + rows[p, 1, i, ks] * w1
                        + rows[p, 2, i, ks] * w2 + rows[p, 3, i, ks] * w3)
            return 0

        lax.fori_loop(0, _CHUNK // _LANES, blend, 0)
        pltpu.async_copy(outbuf.at[p], out_hbm.at[pl.ds(base, _CHUNK)],
                         osem[p])

    ntot = chunks_per_w
    nhalf = ntot // 2

    stage(0, 0)
    stage(1, 1)
    finish(0, 0, drain=False)
    stage(0, 2)
    finish(1, 1, drain=False)

    def loop_body(g2, _):
        ge = 2 * g2
        stage(1, ge + 1)
        finish(0, ge, drain=True)
        stage(0, ge + 2)
        finish(1, ge + 1, drain=True)
        return 0

    lax.fori_loop(1, nhalf - 1, loop_body, 0)

    stage(1, ntot - 1)
    finish(0, ntot - 2, drain=True)
    finish(1, ntot - 1, drain=True)
    pltpu.make_async_copy(outbuf.at[0],
                          out_hbm.at[pl.ds(first * _CHUNK, _CHUNK)],
                          osem0).wait()
    pltpu.make_async_copy(outbuf.at[1],
                          out_hbm.at[pl.ds(first * _CHUNK, _CHUNK)],
                          osem1).wait()


@functools.partial(jax.jit, static_argnames=("chunks_per_w",))
def _sc_sample(xg, yg, table, chunks_per_w):
    n_pad = chunks_per_w * _NW * _CHUNK
    mesh = plsc.VectorSubcoreMesh(core_axis_name="c", subcore_axis_name="s",
                                  num_cores=_NC, num_subcores=_NS)
    return pl.kernel(
        functools.partial(_sc_body, chunks_per_w),
        out_type=jax.ShapeDtypeStruct((n_pad, _FDIM), jnp.float32),
        mesh=mesh,
        compiler_params=pltpu.CompilerParams(use_tc_tiling_on_sc=False),
        scratch_types=[
            pltpu.VMEM((2, _CHUNK), jnp.float32),
            pltpu.VMEM((2, _CHUNK), jnp.float32),
            pltpu.VMEM((2, 4, _CHUNK), jnp.int32),
            pltpu.VMEM((2, 4, _CHUNK), jnp.float32),
            pltpu.VMEM((2, 4, _CHUNK, _FDIM), jnp.float32),
            pltpu.VMEM((2, _CHUNK, _FDIM), jnp.float32),
            pltpu.SemaphoreType.DMA,
            pltpu.SemaphoreType.DMA,
            pltpu.SemaphoreType.DMA,
            pltpu.SemaphoreType.DMA,
        ],
    )(xg, yg, table)


def kernel(x, fm):
    n = x.shape[0]
    chunks_per_w = math.ceil(n / (_NW * _CHUNK))
    chunks_per_w += chunks_per_w % 2
    chunks_per_w = max(chunks_per_w, 4)
    n_pad = chunks_per_w * _NW * _CHUNK
    table = fm[0].reshape(_FDIM, _GRID * _GRID).T
    xp = jnp.pad(x, ((0, n_pad - n), (0, 0)))
    out = _sc_sample(xp[:, 0], xp[:, 1], table, chunks_per_w)
    return out[:n]


# speedup vs baseline: 1.6306x; 1.2830x over previous
import functools
import math

import jax
import jax.numpy as jnp
from jax import lax
from jax.experimental import pallas as pl
from jax.experimental.pallas import tpu as pltpu, tpu_sc as plsc

_FDIM = 64
_GRID = 513
_LANES = 16
_NC = 2
_NS = 16
_NW = _NC * _NS
_CHUNK = 128


def _sc_body(per_w, ntot, tail, xg_hbm, yg_hbm, tab_hbm, out_hbm,
             xbuf, ybuf, idxbuf, wbuf, rows, outbuf,
             gsem0, gsem1, osem0, osem1):
    cid = lax.axis_index("c")
    sid = lax.axis_index("s")
    wid = sid * _NC + cid
    first_x = wid * (ntot * _CHUNK)
    first_o = wid * per_w
    gsem = (gsem0, gsem1)
    osem = (osem0, osem1)

    def stage(p, g):
        xbase = first_x + g * _CHUNK
        pltpu.sync_copy(xg_hbm.at[pl.ds(xbase, _CHUNK)], xbuf.at[p])
        pltpu.sync_copy(yg_hbm.at[pl.ds(xbase, _CHUNK)], ybuf.at[p])
        for j in range(_CHUNK // _LANES):
            sl = pl.ds(_LANES * j, _LANES)
            gx = xbuf[p, sl]
            gy = ybuf[p, sl]
            ix = ((gx + 1.0) * float(_GRID) - 1.0) * 0.5
            iy = ((gy + 1.0) * float(_GRID) - 1.0) * 0.5
            x0 = (ix + 1.0).astype(jnp.int32) - 1
            y0 = (iy + 1.0).astype(jnp.int32) - 1
            wx1 = ix - x0.astype(jnp.float32)
            wx0 = 1.0 - wx1
            wy1 = iy - y0.astype(jnp.float32)
            wy0 = 1.0 - wy1
            wx0 = jnp.where(x0 >= 0, wx0, 0.0)
            wx1 = jnp.where(x0 <= _GRID - 2, wx1, 0.0)
            wy0 = jnp.where(y0 >= 0, wy0, 0.0)
            wy1 = jnp.where(y0 <= _GRID - 2, wy1, 0.0)
            xc0 = jnp.maximum(x0, 0)
            xc1 = jnp.minimum(x0 + 1, _GRID - 1)
            r0 = jnp.maximum(y0, 0) * _GRID
            r1 = jnp.minimum(y0 + 1, _GRID - 1) * _GRID
            idxbuf[p, 0, sl] = r0 + xc0
            idxbuf[p, 1, sl] = r0 + xc1
            idxbuf[p, 2, sl] = r1 + xc0
            idxbuf[p, 3, sl] = r1 + xc1
            wbuf[p, 0, sl] = wx0 * wy0
            wbuf[p, 1, sl] = wx1 * wy0
            wbuf[p, 2, sl] = wx0 * wy1
            wbuf[p, 3, sl] = wx1 * wy1
        for cc in range(4):
            pltpu.async_copy(tab_hbm.at[idxbuf.at[p, cc]], rows.at[p, cc],
                             gsem[p])

    def finish(p, g, drain, nrows=_CHUNK):
        obase = first_o + g * _CHUNK
        for cc in range(4):
            pltpu.make_async_copy(tab_hbm.at[idxbuf.at[p, cc]],
                                  rows.at[p, cc], gsem[p]).wait()

        if drain:
            pltpu.make_async_copy(outbuf.at[p],
                                  out_hbm.at[pl.ds(first_o, _CHUNK)],
                                  osem[p]).wait()

        def blend(g2, _):
            i0 = _LANES * g2
            wv = [wbuf[p, cc, pl.ds(i0, _LANES)] for cc in range(4)]
            for l in range(_LANES):
                i = i0 + l
                w0, w1, w2, w3 = wv[0][l], wv[1][l], wv[2][l], wv[3][l]
                for k in range(_FDIM // _LANES):
                    ks = pl.ds(_LANES * k, _LANES)
                    outbuf[p, i, ks] = (
                        rows[p, 0, i, ks] * w0 + rows[p, 1, i, ks] * w1
                        + rows[p, 2, i, ks] * w2 + rows[p, 3, i, ks] * w3)
            return 0

        lax.fori_loop(0, _CHUNK // _LANES, blend, 0)
        pltpu.async_copy(outbuf.at[p, pl.ds(0, nrows)],
                         out_hbm.at[pl.ds(obase, nrows)], osem[p])

    stage(0, 0)
    stage(1, 1)
    finish(0, 0, drain=False)
    stage(0, 2)
    finish(1, 1, drain=False)

    def loop_body(g2, _):
        ge = 2 * g2
        stage(1, ge + 1)
        finish(0, ge, drain=True)
        stage(0, ge + 2)
        finish(1, ge + 1, drain=True)
        return 0

    if ntot % 2:
        lax.fori_loop(1, (ntot - 1) // 2, loop_body, 0)
        finish(0, ntot - 1, drain=True, nrows=tail)
        last0, last1 = tail, _CHUNK
    else:
        lax.fori_loop(1, (ntot - 2) // 2, loop_body, 0)
        stage(1, ntot - 1)
        finish(0, ntot - 2, drain=True)
        finish(1, ntot - 1, drain=True, nrows=tail)
        last0, last1 = _CHUNK, tail

    pltpu.make_async_copy(outbuf.at[0, pl.ds(0, last0)],
                          out_hbm.at[pl.ds(first_o, last0)], osem0).wait()
    pltpu.make_async_copy(outbuf.at[1, pl.ds(0, last1)],
                          out_hbm.at[pl.ds(first_o, last1)], osem1).wait()


@functools.partial(jax.jit, static_argnames=("n", "per_w", "ntot", "tail"))
def _sc_sample(xg, yg, table, n, per_w, ntot, tail):
    mesh = plsc.VectorSubcoreMesh(core_axis_name="c", subcore_axis_name="s",
                                  num_cores=_NC, num_subcores=_NS)
    return pl.kernel(
        functools.partial(_sc_body, per_w, ntot, tail),
        out_type=jax.ShapeDtypeStruct((n, _FDIM), jnp.float32),
        mesh=mesh,
        compiler_params=pltpu.CompilerParams(use_tc_tiling_on_sc=False),
        scratch_types=[
            pltpu.VMEM((2, _CHUNK), jnp.float32),
            pltpu.VMEM((2, _CHUNK), jnp.float32),
            pltpu.VMEM((2, 4, _CHUNK), jnp.int32),
            pltpu.VMEM((2, 4, _CHUNK), jnp.float32),
            pltpu.VMEM((2, 4, _CHUNK, _FDIM), jnp.float32),
            pltpu.VMEM((2, _CHUNK, _FDIM), jnp.float32),
            pltpu.SemaphoreType.DMA,
            pltpu.SemaphoreType.DMA,
            pltpu.SemaphoreType.DMA,
            pltpu.SemaphoreType.DMA,
        ],
    )(xg, yg, table)


def kernel(x, fm):
    n = x.shape[0]
    assert n % _NW == 0, "point count must split evenly across the 32 tiles"
    per_w = n // _NW
    ntot = math.ceil(per_w / _CHUNK)
    assert ntot >= 3, "pipeline needs at least 3 chunks per tile"
    tail = per_w - (ntot - 1) * _CHUNK
    per_w_pad = ntot * _CHUNK
    table = fm[0].reshape(_FDIM, _GRID * _GRID).T
    xr = x.reshape(_NW, per_w, 2)
    xp = jnp.pad(xr, ((0, 0), (0, per_w_pad - per_w), (0, 0)))
    xp = xp.reshape(_NW * per_w_pad, 2)
    return _sc_sample(xp[:, 0], xp[:, 1], table, n, per_w, ntot, tail)
